# Initial kernel scaffold; baseline (speedup 1.0000x reference)
import jax
import jax.numpy as jnp
from jax.experimental import pallas as pl


def kernel(x, edge_index, W1, b1, gamma1, beta1, W2, b2, gamma2, beta2):
    raise NotImplementedError("write your pallas kernel here")


# speedup vs baseline: 3.2190x; 3.2190x over previous
import functools

import jax
import jax.numpy as jnp
from jax import lax
from jax.experimental import pallas as pl
from jax.experimental.pallas import tpu as pltpu
from jax.experimental.pallas import tpu_sc as plsc

N = 10000
E = 160000
D_IN = 256
D_H = 256
D_L = 128
EPS = 1e-5

LANES = 128
EPAD = 163840
NPAD = 10240
NT = NPAD // 16
KB = 8


def _sc_segsum(rows_per_core: int):
    rt = rows_per_core // 16
    nb = rt // KB
    assert rt * 16 == rows_per_core and nb * KB == rt

    mesh = plsc.VectorSubcoreMesh(core_axis_name="c", subcore_axis_name="s")

    @functools.partial(
        pl.kernel,
        out_type=jax.ShapeDtypeStruct((2, NPAD, LANES), jnp.float32),
        mesh=mesh,
        scratch_types=[
            pltpu.VMEM((KB, LANES), jnp.int32),
            pltpu.VMEM((KB, LANES), jnp.int32),
            pltpu.VMEM((LANES, LANES), jnp.float32),
            pltpu.VMEM_SHARED((NPAD, LANES), jnp.float32),
            pltpu.SemaphoreType.DMA,
        ],
    )
    def k(table, srci, dsti, out, src_v, dst_v, rows_v, acc, sem):
        c = lax.axis_index("c")
        s = lax.axis_index("s")

        z16 = jnp.zeros((16,), jnp.float32)

        def zrow(i, _):
            for j in range(8):
                rows_v[i, pl.ds(j * 16, 16)] = z16
            return _

        lax.fori_loop(0, LANES, zrow, None)
        for b in range(NT // LANES):
            pltpu.sync_copy(rows_v, acc.at[pl.ds(s * NT + b * LANES, LANES)])
        plsc.subcore_barrier()

        def blk(g, _):
            row0 = s * rt + g * KB
            pltpu.sync_copy(srci.at[c, pl.ds(row0, KB)], src_v)
            pltpu.sync_copy(dsti.at[c, pl.ds(row0, KB)], dst_v)
            for j in range(KB):
                pltpu.async_copy(table.at[src_v.at[j]], rows_v, sem).wait()
                pltpu.sync_copy(rows_v, acc.at[dst_v.at[j]], add=True)
            return _

        lax.fori_loop(0, nb, blk, None)
        plsc.subcore_barrier()

        pltpu.sync_copy(acc.at[pl.ds(s * NT, NT)], out.at[c, pl.ds(s * NT, NT)])

    return k


def _tc_matmul_split(x, w):
    bm = 1000
    gi = N // bm

    def body(x_ref, w_ref, o_ref):
        o_ref[...] = jnp.dot(x_ref[...], w_ref[...],
                             preferred_element_type=jnp.float32)

    return pl.pallas_call(
        body,
        grid=(2, gi),
        in_specs=[
            pl.BlockSpec((bm, D_IN), lambda h, i: (i, 0)),
            pl.BlockSpec((D_IN, D_L), lambda h, i: (0, h)),
        ],
        out_specs=pl.BlockSpec((bm, D_L), lambda h, i: (h * gi + i, 0)),
        out_shape=jax.ShapeDtypeStruct((2 * N, D_L), jnp.float32),
    )(x, w)


def _tc_bn_matmul(agg, b1, gamma1, beta1, w2):
    def body(a_ref, b_ref, g_ref, be_ref, w_ref, o_ref):
        h = jnp.concatenate([a_ref[0, :N, :], a_ref[1, :N, :]], axis=1)
        h = h + b_ref[...]
        mean = jnp.mean(h, axis=0, keepdims=True)
        hc = h - mean
        var = jnp.mean(hc * hc, axis=0, keepdims=True)
        hn = hc * lax.rsqrt(var + EPS) * g_ref[...] + be_ref[...]
        o_ref[...] = jnp.dot(hn, w_ref[...], preferred_element_type=jnp.float32)

    return pl.pallas_call(
        body,
        out_shape=jax.ShapeDtypeStruct((N, D_L), jnp.float32),
    )(agg, b1.reshape(1, D_H), gamma1.reshape(1, D_H), beta1.reshape(1, D_H), w2)


def _tc_add_bn(parts, b2, gamma2, beta2):
    def body(p_ref, b_ref, g_ref, be_ref, o_ref):
        h = p_ref[0, :N, :] + p_ref[1, :N, :] + b_ref[...]
        mean = jnp.mean(h, axis=0, keepdims=True)
        hc = h - mean
        var = jnp.mean(hc * hc, axis=0, keepdims=True)
        o_ref[...] = hc * lax.rsqrt(var + EPS) * g_ref[...] + be_ref[...]

    return pl.pallas_call(
        body,
        out_shape=jax.ShapeDtypeStruct((N, D_L), jnp.float32),
    )(parts, b2.reshape(1, D_L), gamma2.reshape(1, D_L), beta2.reshape(1, D_L))


def kernel(x, edge_index, W1, b1, gamma1, beta1, W2, b2, gamma2, beta2):
    src = edge_index[0]
    dst = edge_index[1]

    pad = EPAD - E
    srcp = jnp.concatenate([src, jnp.zeros((pad,), jnp.int32)])
    dstp = jnp.concatenate([dst, jnp.full((pad,), N, jnp.int32)])

    l1_src = jnp.stack([srcp, srcp + N]).reshape(2, EPAD // LANES, LANES)
    l1_dst = jnp.stack([dstp, dstp]).reshape(2, EPAD // LANES, LANES)
    l2_src = srcp.reshape(2, EPAD // (2 * LANES), LANES)
    l2_dst = dstp.reshape(2, EPAD // (2 * LANES), LANES)

    y1 = _tc_matmul_split(x, W1)
    agg1 = _sc_segsum(EPAD // LANES)(y1, l1_src, l1_dst)
    t2 = _tc_bn_matmul(agg1, b1, gamma1, beta1, W2)
    parts = _sc_segsum(EPAD // (2 * LANES))(t2, l2_src, l2_dst)
    return _tc_add_bn(parts, b2, gamma2, beta2)


# speedup vs baseline: 4.4999x; 1.3979x over previous
import functools

import jax
import jax.numpy as jnp
from jax import lax
from jax.experimental import pallas as pl
from jax.experimental.pallas import tpu as pltpu
from jax.experimental.pallas import tpu_sc as plsc

N = 10000
E = 160000
D_IN = 256
D_H = 256
D_L = 128
EPS = 1e-5

LANES = 128
CHUNK = 128
EPAD = 163840
NPAD = 10240
NT = NPAD // 16
QT = 4
SLOTS = 2


def _sc_segsum(rows_core0: int, rows_core1: int):
    rts = (rows_core0 // 16, rows_core1 // 16)
    for rows, rt in zip((rows_core0, rows_core1), rts):
        ntt = rt // QT
        assert rt * 16 == rows and ntt * QT == rt and ntt % 2 == 0

    mesh = plsc.VectorSubcoreMesh(core_axis_name="c", subcore_axis_name="s")

    @functools.partial(
        pl.kernel,
        out_type=jax.ShapeDtypeStruct((2, NPAD, LANES), jnp.float32),
        mesh=mesh,
        scratch_types=[
            pltpu.VMEM((2, QT, CHUNK), jnp.int32),
            pltpu.VMEM((2, QT, CHUNK), jnp.int32),
            [pltpu.VMEM((CHUNK, LANES), jnp.float32) for _ in range(SLOTS)],
            pltpu.VMEM_SHARED((NPAD, LANES), jnp.float32),
            [pltpu.SemaphoreType.DMA for _ in range(SLOTS)],
            [pltpu.SemaphoreType.DMA for _ in range(SLOTS)],
            pltpu.SemaphoreType.DMA,
        ],
    )
    def k(table, srci, dsti, out, src_v, dst_v, rows, acc, gsem, ssem, isem):
        c = lax.axis_index("c")
        s = lax.axis_index("s")
        rt = jnp.where(c == 0, rts[0], rts[1])
        ntt = rt // QT
        base = s * rt

        z16 = jnp.zeros((16,), jnp.float32)

        def zrow(i, _):
            for j in range(8):
                rows[0][i, pl.ds(j * 16, 16)] = z16
            return _

        lax.fori_loop(0, CHUNK, zrow, None)
        for b in range(NT // CHUNK):
            pltpu.sync_copy(rows[0], acc.at[pl.ds(s * NT + b * CHUNK, CHUNK)])
        pltpu.sync_copy(srci.at[c, pl.ds(base, QT)], src_v.at[0])
        pltpu.sync_copy(dsti.at[c, pl.ds(base, QT)], dst_v.at[0])
        plsc.subcore_barrier()

        def gather(ib, q, b):
            pltpu.async_copy(table.at[src_v.at[ib].at[q]], rows[b], gsem[b])

        def gather_wait(b):
            pltpu.make_async_copy(
                table.at[src_v.at[0].at[0]], rows[b], gsem[b]).wait()

        def scatter(ib, q, b):
            pltpu.async_copy(
                rows[b], acc.at[dst_v.at[ib].at[q]], ssem[b], add=True)

        def scatter_wait(b):
            pltpu.make_async_copy(
                rows[b], acc.at[dst_v.at[0].at[0]], ssem[b]).wait()

        def idx_wait(ib):
            pltpu.make_async_copy(
                srci.at[c, pl.ds(base, QT)], src_v.at[ib], isem).wait()
            pltpu.make_async_copy(
                dsti.at[c, pl.ds(base, QT)], dst_v.at[ib], isem).wait()

        for b in range(SLOTS - 1):
            gather(0, b, b)

        def turn(tt, cur):
            nxt = 1 - cur
            for q in range(QT):
                j = tt * QT + q
                b = q % 2
                gather_wait(b)
                scatter(cur, q, b)

                @pl.when(j >= 1)
                def _():
                    scatter_wait(1 - b)

                if q == 0:
                    @pl.when(tt < ntt - 1)
                    def _():
                        nb = base + (tt + 1) * QT
                        pltpu.async_copy(
                            srci.at[c, pl.ds(nb, QT)], src_v.at[nxt], isem)
                        pltpu.async_copy(
                            dsti.at[c, pl.ds(nb, QT)], dst_v.at[nxt], isem)

                if q < QT - 1:
                    gather(cur, q + 1, 1 - b)
                else:
                    @pl.when(tt < ntt - 1)
                    def _():
                        idx_wait(nxt)
                        gather(nxt, 0, 1 - b)

        def two_turns(t2, _):
            turn(2 * t2, 0)
            turn(2 * t2 + 1, 1)
            return _

        lax.fori_loop(0, ntt // 2, two_turns, None)
        scatter_wait(SLOTS - 1)
        plsc.subcore_barrier()

        pltpu.sync_copy(acc.at[pl.ds(s * NT, NT)], out.at[c, pl.ds(s * NT, NT)])

    return k


def _tc_matmul_split(x, w):
    bm = 1000
    gi = N // bm

    def body(x_ref, w_ref, o_ref):
        o_ref[...] = jnp.dot(x_ref[...], w_ref[...],
                             preferred_element_type=jnp.float32)

    return pl.pallas_call(
        body,
        grid=(2, gi),
        in_specs=[
            pl.BlockSpec((bm, D_IN), lambda h, i: (i, 0)),
            pl.BlockSpec((D_IN, D_L), lambda h, i: (0, h)),
        ],
        out_specs=pl.BlockSpec((bm, D_L), lambda h, i: (h * gi + i, 0)),
        out_shape=jax.ShapeDtypeStruct((2 * N, D_L), jnp.float32),
    )(x, w)


def _tc_bn_matmul(agg, b1, gamma1, beta1, w2):
    def body(a_ref, b_ref, g_ref, be_ref, w_ref, o_ref):
        h = jnp.concatenate([a_ref[0, :N, :], a_ref[1, :N, :]], axis=1)
        h = h + b_ref[...]
        mean = jnp.mean(h, axis=0, keepdims=True)
        hc = h - mean
        var = jnp.mean(hc * hc, axis=0, keepdims=True)
        hn = hc * lax.rsqrt(var + EPS) * g_ref[...] + be_ref[...]
        o_ref[...] = jnp.dot(hn, w_ref[...], preferred_element_type=jnp.float32)

    return pl.pallas_call(
        body,
        out_shape=jax.ShapeDtypeStruct((N, D_L), jnp.float32),
    )(agg, b1.reshape(1, D_H), gamma1.reshape(1, D_H), beta1.reshape(1, D_H), w2)


def _tc_add_bn(parts, b2, gamma2, beta2):
    def body(p_ref, b_ref, g_ref, be_ref, o_ref):
        h = p_ref[0, :N, :] + p_ref[1, :N, :] + b_ref[...]
        mean = jnp.mean(h, axis=0, keepdims=True)
        hc = h - mean
        var = jnp.mean(hc * hc, axis=0, keepdims=True)
        o_ref[...] = hc * lax.rsqrt(var + EPS) * g_ref[...] + be_ref[...]

    return pl.pallas_call(
        body,
        out_shape=jax.ShapeDtypeStruct((N, D_L), jnp.float32),
    )(parts, b2.reshape(1, D_L), gamma2.reshape(1, D_L), beta2.reshape(1, D_L))


def kernel(x, edge_index, W1, b1, gamma1, beta1, W2, b2, gamma2, beta2):
    src = edge_index[0]
    dst = edge_index[1]

    pad = EPAD - E
    srcp = jnp.concatenate([src, jnp.zeros((pad,), jnp.int32)])
    dstp = jnp.concatenate([dst, jnp.full((pad,), N, jnp.int32)])

    rows_all = EPAD // CHUNK
    sh1 = (2, rows_all, CHUNK)
    l1_src = jnp.stack([srcp, srcp + N]).reshape(sh1)
    l1_dst = jnp.stack([dstp, dstp]).reshape(sh1)
    r2_0 = rows_all * 9 // 10
    r2_1 = rows_all - r2_0
    src_rows = srcp.reshape(rows_all, CHUNK)
    dst_rows = dstp.reshape(rows_all, CHUNK)
    padr = jnp.zeros((r2_0 - r2_1, CHUNK), jnp.int32)
    l2_src = jnp.stack(
        [src_rows[:r2_0], jnp.concatenate([src_rows[r2_0:], padr])])
    l2_dst = jnp.stack(
        [dst_rows[:r2_0], jnp.concatenate([dst_rows[r2_0:], padr])])

    y1 = _tc_matmul_split(x, W1)
    agg1 = _sc_segsum(rows_all, rows_all)(y1, l1_src, l1_dst)
    t2 = _tc_bn_matmul(agg1, b1, gamma1, beta1, W2)
    parts = _sc_segsum(r2_0, r2_1)(t2, l2_src, l2_dst)
    return _tc_add_bn(parts, b2, gamma2, beta2)
